# baseline (speedup 1.0000x reference)
import functools

import jax
import jax.numpy as jnp
from jax import lax
from jax.experimental import pallas as pl
from jax.experimental.pallas import tpu as pltpu
from jax.experimental.pallas import tpu_sc as plsc

VOCAB = 1000000
EMB = 100
HID = 64
OUT = 5
BATCH = 4096
SEQ = 50

NC, NS = 2, 16
NW = NC * NS
ROWS_PW = BATCH // NW
RPC = 2
NCHUNK = ROWS_PW // RPC
NIDX = RPC * SEQ + 4
EMBP = 128

_WIDEN_BLK = 2048

_mesh = plsc.VectorSubcoreMesh(
    core_axis_name="c", subcore_axis_name="s", num_cores=NC, num_subcores=NS
)


def _widen_body(t_ref, o_ref):
    blk = t_ref[...].T
    o_ref[...] = jnp.concatenate(
        [blk, jnp.zeros((blk.shape[0], EMBP - EMB), jnp.float32)], axis=1)


def _widen_table(table_t):
    return pl.pallas_call(
        _widen_body,
        grid=(pl.cdiv(VOCAB, _WIDEN_BLK),),
        in_specs=[pl.BlockSpec((EMB, _WIDEN_BLK), lambda i: (0, i))],
        out_specs=pl.BlockSpec((_WIDEN_BLK, EMBP), lambda i: (i, 0)),
        out_shape=jax.ShapeDtypeStruct((VOCAB, EMBP), jnp.float32),
    )(table_t)


@functools.partial(
    pl.kernel,
    out_type=jax.ShapeDtypeStruct((BATCH, EMBP), jnp.float32),
    mesh=_mesh,
    scratch_types=[
        pltpu.VMEM((NCHUNK, EMBP), jnp.int32),
        [pltpu.VMEM((NIDX, EMBP), jnp.float32)] * 4,
        pltpu.VMEM((ROWS_PW, EMBP), jnp.float32),
        [pltpu.SemaphoreType.DMA] * 4,
    ],
    compiler_params=pltpu.CompilerParams(use_tc_tiling_on_sc=False),
)
def _pool_kernel(x_hbm, table_hbm, out_hbm, idx_v, bufs, pooled, sems):
    s = lax.axis_index("s")
    c = lax.axis_index("c")
    w = s * NC + c

    pltpu.sync_copy(x_hbm.at[w], idx_v)

    def reduce_chunk(q, buf):
        for rb in range(RPC):
            accs = tuple(
                buf[rb * SEQ, pl.ds(col, 16)] for col in range(0, EMBP, 16))

            def body(t, accs):
                j = rb * SEQ + 1 + t * 7
                for u in range(7):
                    accs = tuple(a + buf[j + u, pl.ds(col, 16)]
                                 for a, col in zip(accs, range(0, EMBP, 16)))
                return accs

            accs = lax.fori_loop(0, 7, body, accs)
            for a, col in zip(accs, range(0, EMBP, 16)):
                pooled[q * RPC + rb, pl.ds(col, 16)] = a

    def gather(q, p):
        return pltpu.async_copy(
            table_hbm.at[idx_v.at[q, pl.ds(0, NIDX)]], bufs[p], sems[p])

    def wait(q, p):
        pltpu.make_async_copy(
            table_hbm.at[idx_v.at[q, pl.ds(0, NIDX)]], bufs[p], sems[p]).wait()

    for p in range(4):
        gather(p, p)

    def step(i, carry):
        q = 4 * i
        for p in range(4):
            wait(q + p, p)
            reduce_chunk(q + p, bufs[p])
            gather(q + p + 4, p)
        return carry

    lax.fori_loop(0, NCHUNK // 4 - 1, step, 0)

    q = NCHUNK - 4
    for p in range(4):
        wait(q + p, p)
        reduce_chunk(q + p, bufs[p])

    pltpu.sync_copy(pooled, out_hbm.at[pl.ds(w * ROWS_PW, ROWS_PW)])


def _mlp_body(p_ref, w1_ref, b1_ref, w2_ref, b2_ref, o_ref):
    h = jnp.dot(p_ref[...], w1_ref[...], preferred_element_type=jnp.float32)
    h = jnp.maximum(h + b1_ref[...], 0.0)
    logits = jnp.dot(h, w2_ref[...], preferred_element_type=jnp.float32)
    logits = logits + b2_ref[...]
    m = jnp.max(logits, axis=1, keepdims=True)
    e = jnp.exp(logits - m)
    o_ref[...] = e / jnp.sum(e, axis=1, keepdims=True)


def kernel(x, table, W1, b1, W2, b2):
    xr = x.astype(jnp.int32).reshape(NW, NCHUNK, RPC * SEQ)
    xw = jnp.pad(xr, ((0, 0), (0, 0), (0, EMBP - RPC * SEQ)))

    table_p = _widen_table(table.T)
    pooled_sum = _pool_kernel(xw, table_p)

    w1s = jnp.pad((W1 * (1.0 / SEQ)).T, ((0, EMBP - EMB), (0, 0)))
    b1r = b1.reshape(1, HID)
    w2t = W2.T
    b2r = b2.reshape(1, OUT)

    return pl.pallas_call(
        _mlp_body,
        out_shape=jax.ShapeDtypeStruct((BATCH, OUT), jnp.float32),
    )(pooled_sum, w1s, b1r, w2t, b2r)


# speedup vs baseline: 1.2906x; 1.2906x over previous
import functools

import jax
import jax.numpy as jnp
from jax import lax
from jax.experimental import pallas as pl
from jax.experimental.pallas import tpu as pltpu
from jax.experimental.pallas import tpu_sc as plsc

VOCAB = 1000000
EMB = 100
HID = 64
OUT = 5
BATCH = 4096
SEQ = 50

NC, NS = 2, 16
NW = NC * NS
ROWS_PW = BATCH // NW
RPC = 2
NCHUNK = ROWS_PW // RPC
NIDX = RPC * SEQ + 4
EMBP = 128

_WIDEN_BLK = 8192

_mesh = plsc.VectorSubcoreMesh(
    core_axis_name="c", subcore_axis_name="s", num_cores=NC, num_subcores=NS
)


def _widen_body(t_ref, o_ref):
    blk = t_ref[...].T
    o_ref[...] = jnp.concatenate(
        [blk, jnp.zeros((blk.shape[0], EMBP - EMB), jnp.float32)], axis=1)


def _widen_table(table_t):
    return pl.pallas_call(
        _widen_body,
        grid=(pl.cdiv(VOCAB, _WIDEN_BLK),),
        in_specs=[pl.BlockSpec((EMB, _WIDEN_BLK), lambda i: (0, i))],
        out_specs=pl.BlockSpec((_WIDEN_BLK, EMBP), lambda i: (i, 0)),
        out_shape=jax.ShapeDtypeStruct((VOCAB, EMBP), jnp.float32),
    )(table_t)


@functools.partial(
    pl.kernel,
    out_type=jax.ShapeDtypeStruct((BATCH, EMBP), jnp.float32),
    mesh=_mesh,
    scratch_types=[
        pltpu.VMEM((NCHUNK, EMBP), jnp.int32),
        [pltpu.VMEM((NIDX, EMBP), jnp.float32)] * 4,
        pltpu.VMEM((ROWS_PW, EMBP), jnp.float32),
        [pltpu.SemaphoreType.DMA] * 4,
    ],
    compiler_params=pltpu.CompilerParams(use_tc_tiling_on_sc=False),
)
def _pool_kernel(x_hbm, table_hbm, out_hbm, idx_v, bufs, pooled, sems):
    s = lax.axis_index("s")
    c = lax.axis_index("c")
    w = s * NC + c

    pltpu.sync_copy(x_hbm.at[w], idx_v)

    def reduce_chunk(q, buf):
        for rb in range(RPC):
            accs = tuple(
                buf[rb * SEQ, pl.ds(col, 16)] for col in range(0, EMBP, 16))

            def body(t, accs):
                j = rb * SEQ + 1 + t * 7
                for u in range(7):
                    accs = tuple(a + buf[j + u, pl.ds(col, 16)]
                                 for a, col in zip(accs, range(0, EMBP, 16)))
                return accs

            accs = lax.fori_loop(0, 7, body, accs)
            for a, col in zip(accs, range(0, EMBP, 16)):
                pooled[q * RPC + rb, pl.ds(col, 16)] = a

    def gather(q, p):
        return pltpu.async_copy(
            table_hbm.at[idx_v.at[q, pl.ds(0, NIDX)]], bufs[p], sems[p])

    def wait(q, p):
        pltpu.make_async_copy(
            table_hbm.at[idx_v.at[q, pl.ds(0, NIDX)]], bufs[p], sems[p]).wait()

    for p in range(4):
        gather(p, p)

    def step(i, carry):
        q = 4 * i
        for p in range(4):
            wait(q + p, p)
            reduce_chunk(q + p, bufs[p])
            gather(q + p + 4, p)
        return carry

    lax.fori_loop(0, NCHUNK // 4 - 1, step, 0)

    q = NCHUNK - 4
    for p in range(4):
        wait(q + p, p)
        reduce_chunk(q + p, bufs[p])

    pltpu.sync_copy(pooled, out_hbm.at[pl.ds(w * ROWS_PW, ROWS_PW)])


def _mlp_body(p_ref, w1_ref, b1_ref, w2_ref, b2_ref, o_ref):
    h = jnp.dot(p_ref[...], w1_ref[...], preferred_element_type=jnp.float32)
    h = jnp.maximum(h + b1_ref[...], 0.0)
    logits = jnp.dot(h, w2_ref[...], preferred_element_type=jnp.float32)
    logits = logits + b2_ref[...]
    m = jnp.max(logits, axis=1, keepdims=True)
    e = jnp.exp(logits - m)
    o_ref[...] = e / jnp.sum(e, axis=1, keepdims=True)


def kernel(x, table, W1, b1, W2, b2):
    xr = x.astype(jnp.int32).reshape(NW, NCHUNK, RPC * SEQ)
    xw = jnp.pad(xr, ((0, 0), (0, 0), (0, EMBP - RPC * SEQ)))

    table_p = _widen_table(table.T)
    pooled_sum = _pool_kernel(xw, table_p)

    w1s = jnp.pad((W1 * (1.0 / SEQ)).T, ((0, EMBP - EMB), (0, 0)))
    b1r = b1.reshape(1, HID)
    w2t = W2.T
    b2r = b2.reshape(1, OUT)

    return pl.pallas_call(
        _mlp_body,
        out_shape=jax.ShapeDtypeStruct((BATCH, OUT), jnp.float32),
    )(pooled_sum, w1s, b1r, w2t, b2r)


# speedup vs baseline: 1.3078x; 1.0133x over previous
import functools

import jax
import jax.numpy as jnp
from jax import lax
from jax.experimental import pallas as pl
from jax.experimental.pallas import tpu as pltpu
from jax.experimental.pallas import tpu_sc as plsc

VOCAB = 1000000
EMB = 100
HID = 64
OUT = 5
BATCH = 4096
SEQ = 50

NC, NS = 2, 16
NW = NC * NS
ROWS_PW = BATCH // NW
RPC = 2
NCHUNK = ROWS_PW // RPC
NIDX = RPC * SEQ + 4
EMBP = 128

_WIDEN_BLK = 16384

_mesh = plsc.VectorSubcoreMesh(
    core_axis_name="c", subcore_axis_name="s", num_cores=NC, num_subcores=NS
)


def _widen_body(t_ref, o_ref):
    blk = t_ref[...].T
    o_ref[...] = jnp.concatenate(
        [blk, jnp.zeros((blk.shape[0], EMBP - EMB), jnp.float32)], axis=1)


def _widen_table(table_t):
    return pl.pallas_call(
        _widen_body,
        grid=(pl.cdiv(VOCAB, _WIDEN_BLK),),
        in_specs=[pl.BlockSpec((EMB, _WIDEN_BLK), lambda i: (0, i))],
        out_specs=pl.BlockSpec((_WIDEN_BLK, EMBP), lambda i: (i, 0)),
        out_shape=jax.ShapeDtypeStruct((VOCAB, EMBP), jnp.float32),
    )(table_t)


@functools.partial(
    pl.kernel,
    out_type=jax.ShapeDtypeStruct((BATCH, EMBP), jnp.float32),
    mesh=_mesh,
    scratch_types=[
        pltpu.VMEM((NCHUNK, EMBP), jnp.int32),
        [pltpu.VMEM((NIDX, EMBP), jnp.float32)] * 4,
        pltpu.VMEM((ROWS_PW, EMBP), jnp.float32),
        [pltpu.SemaphoreType.DMA] * 4,
    ],
    compiler_params=pltpu.CompilerParams(use_tc_tiling_on_sc=False),
)
def _pool_kernel(x_hbm, table_hbm, out_hbm, idx_v, bufs, pooled, sems):
    s = lax.axis_index("s")
    c = lax.axis_index("c")
    w = s * NC + c

    pltpu.sync_copy(x_hbm.at[w], idx_v)

    def reduce_chunk(q, buf):
        for rb in range(RPC):
            accs = tuple(
                buf[rb * SEQ, pl.ds(col, 16)] for col in range(0, EMBP, 16))

            def body(t, accs):
                j = rb * SEQ + 1 + t * 7
                for u in range(7):
                    accs = tuple(a + buf[j + u, pl.ds(col, 16)]
                                 for a, col in zip(accs, range(0, EMBP, 16)))
                return accs

            accs = lax.fori_loop(0, 7, body, accs)
            for a, col in zip(accs, range(0, EMBP, 16)):
                pooled[q * RPC + rb, pl.ds(col, 16)] = a

    def gather(q, p):
        return pltpu.async_copy(
            table_hbm.at[idx_v.at[q, pl.ds(0, NIDX)]], bufs[p], sems[p])

    def wait(q, p):
        pltpu.make_async_copy(
            table_hbm.at[idx_v.at[q, pl.ds(0, NIDX)]], bufs[p], sems[p]).wait()

    for p in range(4):
        gather(p, p)

    def step(i, carry):
        q = 4 * i
        for p in range(4):
            wait(q + p, p)
            reduce_chunk(q + p, bufs[p])
            gather(q + p + 4, p)
        return carry

    lax.fori_loop(0, NCHUNK // 4 - 1, step, 0)

    q = NCHUNK - 4
    for p in range(4):
        wait(q + p, p)
        reduce_chunk(q + p, bufs[p])

    pltpu.sync_copy(pooled, out_hbm.at[pl.ds(w * ROWS_PW, ROWS_PW)])


def _mlp_body(p_ref, w1_ref, b1_ref, w2_ref, b2_ref, o_ref):
    h = jnp.dot(p_ref[...], w1_ref[...], preferred_element_type=jnp.float32)
    h = jnp.maximum(h + b1_ref[...], 0.0)
    logits = jnp.dot(h, w2_ref[...], preferred_element_type=jnp.float32)
    logits = logits + b2_ref[...]
    m = jnp.max(logits, axis=1, keepdims=True)
    e = jnp.exp(logits - m)
    o_ref[...] = e / jnp.sum(e, axis=1, keepdims=True)


def kernel(x, table, W1, b1, W2, b2):
    xr = x.astype(jnp.int32).reshape(NW, NCHUNK, RPC * SEQ)
    xw = jnp.pad(xr, ((0, 0), (0, 0), (0, EMBP - RPC * SEQ)))

    table_p = _widen_table(table.T)
    pooled_sum = _pool_kernel(xw, table_p)

    w1s = jnp.pad((W1 * (1.0 / SEQ)).T, ((0, EMBP - EMB), (0, 0)))
    b1r = b1.reshape(1, HID)
    w2t = W2.T
    b2r = b2.reshape(1, OUT)

    return pl.pallas_call(
        _mlp_body,
        out_shape=jax.ShapeDtypeStruct((BATCH, OUT), jnp.float32),
    )(pooled_sum, w1s, b1r, w2t, b2r)


# speedup vs baseline: 1.3124x; 1.0035x over previous
import functools

import jax
import jax.numpy as jnp
from jax import lax
from jax.experimental import pallas as pl
from jax.experimental.pallas import tpu as pltpu
from jax.experimental.pallas import tpu_sc as plsc

VOCAB = 1000000
EMB = 100
HID = 64
OUT = 5
BATCH = 4096
SEQ = 50

NC, NS = 2, 16
NW = NC * NS
ROWS_PW = BATCH // NW
RPC = 2
NCHUNK = ROWS_PW // RPC
NIDX = RPC * SEQ + 4
EMBP = 128

_WIDEN_BLK = 24576

_mesh = plsc.VectorSubcoreMesh(
    core_axis_name="c", subcore_axis_name="s", num_cores=NC, num_subcores=NS
)


def _widen_body(t_ref, o_ref):
    blk = t_ref[...].T
    o_ref[...] = jnp.concatenate(
        [blk, jnp.zeros((blk.shape[0], EMBP - EMB), jnp.float32)], axis=1)


def _widen_table(table_t):
    return pl.pallas_call(
        _widen_body,
        grid=(pl.cdiv(VOCAB, _WIDEN_BLK),),
        in_specs=[pl.BlockSpec((EMB, _WIDEN_BLK), lambda i: (0, i))],
        out_specs=pl.BlockSpec((_WIDEN_BLK, EMBP), lambda i: (i, 0)),
        out_shape=jax.ShapeDtypeStruct((VOCAB, EMBP), jnp.float32),
    )(table_t)


@functools.partial(
    pl.kernel,
    out_type=jax.ShapeDtypeStruct((BATCH, EMBP), jnp.float32),
    mesh=_mesh,
    scratch_types=[
        pltpu.VMEM((NCHUNK, EMBP), jnp.int32),
        [pltpu.VMEM((NIDX, EMBP), jnp.float32)] * 4,
        pltpu.VMEM((ROWS_PW, EMBP), jnp.float32),
        [pltpu.SemaphoreType.DMA] * 4,
    ],
    compiler_params=pltpu.CompilerParams(use_tc_tiling_on_sc=False),
)
def _pool_kernel(x_hbm, table_hbm, out_hbm, idx_v, bufs, pooled, sems):
    s = lax.axis_index("s")
    c = lax.axis_index("c")
    w = s * NC + c

    pltpu.sync_copy(x_hbm.at[w], idx_v)

    def reduce_chunk(q, buf):
        for rb in range(RPC):
            accs = tuple(
                buf[rb * SEQ, pl.ds(col, 16)] for col in range(0, EMBP, 16))

            def body(t, accs):
                j = rb * SEQ + 1 + t * 7
                for u in range(7):
                    accs = tuple(a + buf[j + u, pl.ds(col, 16)]
                                 for a, col in zip(accs, range(0, EMBP, 16)))
                return accs

            accs = lax.fori_loop(0, 7, body, accs)
            for a, col in zip(accs, range(0, EMBP, 16)):
                pooled[q * RPC + rb, pl.ds(col, 16)] = a

    def gather(q, p):
        return pltpu.async_copy(
            table_hbm.at[idx_v.at[q, pl.ds(0, NIDX)]], bufs[p], sems[p])

    def wait(q, p):
        pltpu.make_async_copy(
            table_hbm.at[idx_v.at[q, pl.ds(0, NIDX)]], bufs[p], sems[p]).wait()

    for p in range(4):
        gather(p, p)

    def step(i, carry):
        q = 4 * i
        for p in range(4):
            wait(q + p, p)
            reduce_chunk(q + p, bufs[p])
            gather(q + p + 4, p)
        return carry

    lax.fori_loop(0, NCHUNK // 4 - 1, step, 0)

    q = NCHUNK - 4
    for p in range(4):
        wait(q + p, p)
        reduce_chunk(q + p, bufs[p])

    pltpu.sync_copy(pooled, out_hbm.at[pl.ds(w * ROWS_PW, ROWS_PW)])


def _mlp_body(p_ref, w1_ref, b1_ref, w2_ref, b2_ref, o_ref):
    h = jnp.dot(p_ref[...], w1_ref[...], preferred_element_type=jnp.float32)
    h = jnp.maximum(h + b1_ref[...], 0.0)
    logits = jnp.dot(h, w2_ref[...], preferred_element_type=jnp.float32)
    logits = logits + b2_ref[...]
    m = jnp.max(logits, axis=1, keepdims=True)
    e = jnp.exp(logits - m)
    o_ref[...] = e / jnp.sum(e, axis=1, keepdims=True)


def kernel(x, table, W1, b1, W2, b2):
    xr = x.astype(jnp.int32).reshape(NW, NCHUNK, RPC * SEQ)
    xw = jnp.pad(xr, ((0, 0), (0, 0), (0, EMBP - RPC * SEQ)))

    table_p = _widen_table(table.T)
    pooled_sum = _pool_kernel(xw, table_p)

    w1s = jnp.pad((W1 * (1.0 / SEQ)).T, ((0, EMBP - EMB), (0, 0)))
    b1r = b1.reshape(1, HID)
    w2t = W2.T
    b2r = b2.reshape(1, OUT)

    return pl.pallas_call(
        _mlp_body,
        out_shape=jax.ShapeDtypeStruct((BATCH, OUT), jnp.float32),
    )(pooled_sum, w1s, b1r, w2t, b2r)
